# baseline (speedup 1.0000x reference)
import functools

import jax
import jax.numpy as jnp
from jax import lax
from jax.experimental import pallas as pl
from jax.experimental.pallas import tpu as pltpu
from jax.experimental.pallas import tpu_sc as plsc

B = 16384
N = 200
NCONT = 3
NTOT = N * NCONT
NW = 32
LANES = 16
GROUP = 2 * LANES
ROWS_PER_W = B // NW
N_GROUPS = ROWS_PER_W // GROUP
UNROLL = 25


def _sev_body(x_hbm, out_hbm, buf_a, buf_b, outb, sem_a, sem_b):
    nc = 2
    wid = lax.axis_index("s") * nc + lax.axis_index("c")
    base_row = wid * ROWS_PER_W

    lane = lax.iota(jnp.int32, LANES)
    zeros = jnp.zeros((LANES,), jnp.float32)
    ones = jnp.ones((LANES,), jnp.float32)
    row_idx = [h * LANES + lane for h in range(2)]
    col_of = [jnp.full((LANES,), k, jnp.int32) for k in range(5)]

    def fetch(g, buf, sem):
        g = jnp.minimum(g, N_GROUPS - 1)
        row0 = base_row + g * GROUP
        for c in range(NCONT):
            pltpu.async_copy(
                x_hbm.at[c, pl.ds(row0, GROUP), :],
                buf.at[:, pl.ds(c * N, N)],
                sem,
            )

    def wait_fetch(buf, sem):
        for c in range(NCONT):
            pltpu.make_async_copy(
                x_hbm.at[0, pl.ds(0, GROUP), :],
                buf.at[:, pl.ds(c * N, N)],
                sem,
            ).wait()

    def compute(g, buf):
        def body(t, carry):
            colv = carry[0]
            dp = [list(carry[1:6]), list(carry[6:11])]
            for _ in range(UNROLL):
                for h in range(2):
                    d0, d1, d2, d3, d4 = dp[h]
                    pi = plsc.load_gather(buf, [row_idx[h], colv])
                    om = 1.0 - pi
                    dp[h] = [
                        d0 * om,
                        d1 * om + d0 * pi,
                        d2 * om + d1 * pi,
                        d3 * om + d2 * pi,
                        d4 * om + d3 * pi,
                    ]
                colv = colv + 1
            return (colv, *dp[0], *dp[1])

        init = (jnp.zeros((LANES,), jnp.int32),
                ones, zeros, zeros, zeros, zeros,
                ones, zeros, zeros, zeros, zeros)
        res = lax.fori_loop(0, NTOT // UNROLL, body, init)
        dp = [res[1:6], res[6:11]]

        for h in range(2):
            d0, d1, d2, d3, d4 = dp[h]
            sev0 = d0
            sev1 = d1 + d2
            sev2 = d3 + d4
            sev3 = 1.0 - (sev0 + sev1 + sev2)
            rows = g * GROUP + h * LANES + lane
            for k, val in enumerate((sev0, sev1, sev2, sev3, zeros)):
                plsc.store_scatter(outb, [rows, col_of[k]], val)

    fetch(0, buf_a, sem_a)
    fetch(1, buf_b, sem_b)

    def outer(gp, _):
        ga = 2 * gp
        wait_fetch(buf_a, sem_a)
        compute(ga, buf_a)
        fetch(ga + 2, buf_a, sem_a)
        wait_fetch(buf_b, sem_b)
        compute(ga + 1, buf_b)
        fetch(ga + 3, buf_b, sem_b)
        return 0

    lax.fori_loop(0, N_GROUPS // 2, outer, 0)
    wait_fetch(buf_a, sem_a)
    wait_fetch(buf_b, sem_b)

    pltpu.sync_copy(outb, out_hbm.at[pl.ds(base_row, ROWS_PER_W)])


@jax.jit
def kernel(x):
    mesh = plsc.VectorSubcoreMesh(core_axis_name="c", subcore_axis_name="s")
    run = functools.partial(
        pl.kernel,
        mesh=mesh,
        out_type=jax.ShapeDtypeStruct((B, 5), jnp.float32),
        scratch_types=[
            pltpu.VMEM((GROUP, NTOT), jnp.float32),
            pltpu.VMEM((GROUP, NTOT), jnp.float32),
            pltpu.VMEM((ROWS_PER_W, 5), jnp.float32),
            pltpu.SemaphoreType.DMA,
            pltpu.SemaphoreType.DMA,
        ],
        compiler_params=pltpu.CompilerParams(
            needs_layout_passes=False, use_tc_tiling_on_sc=False),
    )(_sev_body)
    return run(x)
